# baseline (speedup 1.0000x reference)
import functools

import jax
import jax.numpy as jnp
from jax import lax
from jax.experimental import pallas as pl
from jax.experimental.pallas import tpu as pltpu, tpu_sc as plsc

VOCAB = 1000000
DIM = 64
NB, NS_TOK = 4096, 200

NC, NS = 2, 16
NW = NC * NS
BW = NB // NW


def _body(xt_hbm, table_hbm, out_hbm, xv, rows0, rows1, rt0, rt1,
          gsem0, gsem1, osem0, osem1):
    w = lax.axis_index("s") * NC + lax.axis_index("c")
    b0 = w * BW

    pltpu.sync_copy(xt_hbm.at[:, pl.ds(b0, BW)], xv)

    rows = (rows0, rows1)
    rt = (rt0, rt1)
    gsem = (gsem0, gsem1)
    osem = (osem0, osem1)

    iotas = [lax.iota(jnp.int32, 16) + (16 * k) for k in range(8)]

    def fire(s, buf):
        pltpu.async_copy(table_hbm.at[xv.at[s]], rows[buf], gsem[buf])

    def drain(s, buf):
        pltpu.make_async_copy(table_hbm.at[xv.at[s]], rows[buf], gsem[buf]).wait()

    def store(s, buf):
        pltpu.async_copy(rt[buf], out_hbm.at[s, :, pl.ds(b0, BW)], osem[buf])

    def wait_store(s, buf):
        pltpu.make_async_copy(rt[buf], out_hbm.at[s, :, pl.ds(b0, BW)], osem[buf]).wait()

    def transpose(buf):
        @plsc.parallel_loop(0, DIM, step=1, unroll=8)
        def _(c):
            col = jnp.full((16,), c, jnp.int32)
            for k in range(8):
                v = plsc.load_gather(rows[buf], [iotas[k], col])
                rt[buf][c, pl.ds(16 * k, 16)] = v

    fire(0, 0)

    @pl.loop(0, NS_TOK, step=2)
    def _(s0):
        for b in range(2):
            s = s0 + b
            drain(s, b)

            @pl.when(s + 1 < NS_TOK)
            def _():
                fire(s + 1, 1 - b)

            @pl.when(s >= 2)
            def _():
                wait_store(s - 2, b)

            transpose(b)
            store(s, b)

    wait_store(NS_TOK - 2, 0)
    wait_store(NS_TOK - 1, 1)


@jax.jit
def _lookup(x_t, table_lin):
    mesh = plsc.VectorSubcoreMesh(core_axis_name="c", subcore_axis_name="s")
    k = pl.kernel(
        _body,
        out_type=jax.ShapeDtypeStruct((NS_TOK, DIM, NB), jnp.float32),
        mesh=mesh,
        scratch_types=[
            pltpu.VMEM((NS_TOK, BW), jnp.int32),
            pltpu.VMEM((BW, DIM), jnp.float32),
            pltpu.VMEM((BW, DIM), jnp.float32),
            pltpu.VMEM((DIM, BW), jnp.float32),
            pltpu.VMEM((DIM, BW), jnp.float32),
            pltpu.SemaphoreType.DMA,
            pltpu.SemaphoreType.DMA,
            pltpu.SemaphoreType.DMA,
            pltpu.SemaphoreType.DMA,
        ],
        compiler_params=pltpu.CompilerParams(
            use_tc_tiling_on_sc=False, needs_layout_passes=False
        ),
    )
    return k(x_t, table_lin)


def kernel(x, table):
    x_t = jnp.transpose(x).astype(jnp.int32)
    t_pair = jax.lax.optimization_barrier(jnp.reshape(table, (VOCAB // 2, 2 * DIM)))
    t_lin = jnp.reshape(t_pair, (VOCAB, DIM))
    out_t = _lookup(x_t, t_lin)
    return jnp.transpose(out_t, (2, 0, 1))
